# Initial kernel scaffold; baseline (speedup 1.0000x reference)
import jax
import jax.numpy as jnp
from jax.experimental import pallas as pl


def kernel(model_embeddings, feature_tensors, positive_labels, negative_labels, padding_mask, target_padding_mask, W_items):
    raise NotImplementedError("write your pallas kernel here")


# speedup vs baseline: 3.4702x; 3.4702x over previous
import jax
import jax.numpy as jnp
from jax.experimental import pallas as pl
from jax.experimental.pallas import tpu as pltpu

_B, _S, _P = 16, 50, 4
_V, _D = 100000, 64
_T = _B * _S
_TV = 2048
_NT = (_V + _TV - 1) // _TV


def _bce_body(e_ref, w_ref, lbl_ref, tm_ref, out_ref):
    i = pl.program_id(0)
    e = e_ref[...]
    wt = w_ref[...]
    logits = jax.lax.dot_general(
        e, wt, (((1,), (1,)), ((), ())), preferred_element_type=jnp.float32
    )
    col = jax.lax.broadcasted_iota(jnp.int32, (_T, _TV), 1) + i * _TV
    sp = jnp.maximum(logits, 0.0) + jnp.log1p(jnp.exp(-jnp.abs(logits)))
    lbl = lbl_ref[...]
    y = col == lbl[:, 0:1]
    for p in range(1, _P):
        y = y | (col == lbl[:, p : p + 1])
    per = sp - logits * y.astype(jnp.float32)
    per = jnp.where(col < _V, per, 0.0)
    tm = tm_ref[...]
    w_tok = (jnp.sum(tm, axis=1, keepdims=True) > 0).astype(jnp.float32)
    part = jnp.sum(jnp.sum(per, axis=1, keepdims=True) * w_tok)

    @pl.when(i == 0)
    def _init():
        out_ref[...] = jnp.zeros((1, 1), jnp.float32)

    out_ref[...] += jnp.full((1, 1), part, jnp.float32)

    @pl.when(i == _NT - 1)
    def _fin():
        nv = jnp.maximum(jnp.sum(w_tok), 1.0)
        out_ref[...] = out_ref[...] / nv


def kernel(model_embeddings, feature_tensors, positive_labels, negative_labels, padding_mask, target_padding_mask, W_items):
    e2 = model_embeddings.reshape(_T, _D)
    lbl2 = positive_labels.reshape(_T, _P).astype(jnp.int32)
    tm2 = target_padding_mask.reshape(_T, _P).astype(jnp.float32)
    out = pl.pallas_call(
        _bce_body,
        grid=(_NT,),
        in_specs=[
            pl.BlockSpec((_T, _D), lambda i: (0, 0)),
            pl.BlockSpec((_TV, _D), lambda i: (i, 0)),
            pl.BlockSpec((_T, _P), lambda i: (0, 0)),
            pl.BlockSpec((_T, _P), lambda i: (0, 0)),
        ],
        out_specs=pl.BlockSpec((1, 1), lambda i: (0, 0)),
        out_shape=jax.ShapeDtypeStruct((1, 1), jnp.float32),
        compiler_params=pltpu.CompilerParams(
            dimension_semantics=("arbitrary",),
        ),
    )(e2, W_items, lbl2, tm2)
    return out[0, 0]


# speedup vs baseline: 3.7324x; 1.0756x over previous
import functools

import jax
import jax.numpy as jnp
from jax import lax
from jax.experimental import pallas as pl
from jax.experimental.pallas import tpu as pltpu
from jax.experimental.pallas import tpu_sc as plsc

_B, _S, _P = 16, 50, 4
_V, _D = 100000, 64
_T = _B * _S
_TV = 2048
_NT = (_V + _TV - 1) // _TV
_NC, _NS = 2, 16
_NW = _NC * _NS
_GP = 3328
_BPW = _GP // _NW
_D2 = 2 * _D

_sc_mesh = plsc.VectorSubcoreMesh(core_axis_name="c", subcore_axis_name="s")


@functools.partial(
    pl.kernel,
    mesh=_sc_mesh,
    out_type=jax.ShapeDtypeStruct((_GP, _D2), jnp.float32),
    scratch_types=[
        pltpu.VMEM((_BPW,), jnp.int32),
        pltpu.VMEM((_BPW, _D2), jnp.float32),
        pltpu.SemaphoreType.DMA,
    ],
)
def _sc_gather(table_hbm, idx_hbm, out_hbm, idx_v, rows_v, sem):
    wid = lax.axis_index("s") * _NC + lax.axis_index("c")
    base = wid * _BPW
    pltpu.sync_copy(idx_hbm.at[pl.ds(base, _BPW)], idx_v)
    pltpu.async_copy(table_hbm.at[idx_v], rows_v, sem).wait()
    pltpu.sync_copy(rows_v, out_hbm.at[pl.ds(base, _BPW)])


def _bce_body(e_ref, w_ref, lbl_ref, tm_ref, g_ref, out_ref):
    i = pl.program_id(0)
    e = e_ref[...]
    wt = w_ref[...]
    logits = lax.dot_general(
        e, wt, (((1,), (1,)), ((), ())), preferred_element_type=jnp.float32
    )
    col = lax.broadcasted_iota(jnp.int32, (_T, _TV), 1) + i * _TV
    sp = jnp.maximum(logits, 0.0) + jnp.log1p(jnp.exp(-jnp.abs(logits)))
    sp = jnp.where(col < _V, sp, 0.0)
    tm = tm_ref[...]
    w_tok = (jnp.sum(tm, axis=1, keepdims=True) > 0).astype(jnp.float32)
    part = jnp.sum(jnp.sum(sp, axis=1, keepdims=True) * w_tok)

    @pl.when(i == 0)
    def _init():
        out_ref[...] = jnp.zeros((1, 1), jnp.float32)

    out_ref[...] += jnp.full((1, 1), part, jnp.float32)

    @pl.when(i == _NT - 1)
    def _fin():
        g = g_ref[...]
        lbl = lbl_ref[...]
        corr = jnp.float32(0.0)
        for p in range(_P):
            gp = g[p * _T : (p + 1) * _T, :]
            dots_lo = jnp.sum(gp[:, :_D] * e, axis=1, keepdims=True)
            dots_hi = jnp.sum(gp[:, _D:] * e, axis=1, keepdims=True)
            par = lbl[:, p : p + 1] % 2
            dots = jnp.where(par == 1, dots_hi, dots_lo)
            dp = w_tok
            for q in range(p):
                dp = dp * (lbl[:, p : p + 1] != lbl[:, q : q + 1]).astype(
                    jnp.float32
                )
            corr += jnp.sum(dots * dp)
        nv = jnp.maximum(jnp.sum(w_tok), 1.0)
        out_ref[...] = (out_ref[...] - corr) / nv


def kernel(model_embeddings, feature_tensors, positive_labels, negative_labels, padding_mask, target_padding_mask, W_items):
    e2 = model_embeddings.reshape(_T, _D)
    lbl2 = positive_labels.reshape(_T, _P).astype(jnp.int32)
    tm2 = target_padding_mask.reshape(_T, _P).astype(jnp.float32)
    idx_pm = lbl2.T.reshape(-1)
    idx_pad = jnp.concatenate(
        [idx_pm // 2, jnp.zeros((_GP - _P * _T,), jnp.int32)]
    )
    w_pairs = W_items.reshape(_V // 2, _D2)
    g = _sc_gather(w_pairs, idx_pad)
    out = pl.pallas_call(
        _bce_body,
        grid=(_NT,),
        in_specs=[
            pl.BlockSpec((_T, _D), lambda i: (0, 0)),
            pl.BlockSpec((_TV, _D), lambda i: (i, 0)),
            pl.BlockSpec((_T, _P), lambda i: (0, 0)),
            pl.BlockSpec((_T, _P), lambda i: (0, 0)),
            pl.BlockSpec((_P * _T, _D2), lambda i: (0, 0)),
        ],
        out_specs=pl.BlockSpec((1, 1), lambda i: (0, 0)),
        out_shape=jax.ShapeDtypeStruct((1, 1), jnp.float32),
        compiler_params=pltpu.CompilerParams(
            dimension_semantics=("arbitrary",),
        ),
    )(e2, W_items, lbl2, tm2, g)
    return out[0, 0]


# speedup vs baseline: 4.3383x; 1.1623x over previous
import functools

import jax
import jax.numpy as jnp
from jax import lax
from jax.experimental import pallas as pl
from jax.experimental.pallas import tpu as pltpu
from jax.experimental.pallas import tpu_sc as plsc

_B, _S, _P = 16, 50, 4
_V, _D = 100000, 64
_T = _B * _S
_TV = 2048
_NT = (_V + _TV - 1) // _TV
_NC, _NS = 2, 16
_NW = _NC * _NS
_GP = 3328
_BPW = _GP // _NW
_D2 = 2 * _D

@functools.lru_cache(maxsize=1)
def _make_sc_gather():
    mesh = plsc.VectorSubcoreMesh(core_axis_name="c", subcore_axis_name="s")

    @functools.partial(
        pl.kernel,
        mesh=mesh,
        out_type=jax.ShapeDtypeStruct((_GP, _D2), jnp.float32),
        scratch_types=[
            pltpu.VMEM((_BPW,), jnp.int32),
            pltpu.VMEM((_BPW, _D2), jnp.float32),
            pltpu.SemaphoreType.DMA,
        ],
    )
    def _sc_gather(table_hbm, idx_hbm, out_hbm, idx_v, rows_v, sem):
        wid = lax.axis_index("s") * _NC + lax.axis_index("c")
        base = wid * _BPW
        pltpu.sync_copy(idx_hbm.at[pl.ds(base, _BPW)], idx_v)
        pltpu.async_copy(table_hbm.at[idx_v], rows_v, sem).wait()
        pltpu.sync_copy(rows_v, out_hbm.at[pl.ds(base, _BPW)])

    return _sc_gather


def _bce_body(e_ref, w_ref, lbl_ref, tm_ref, g_ref, out_ref):
    i = pl.program_id(0)
    e = e_ref[...]
    wt = w_ref[...]
    row = lax.broadcasted_iota(jnp.int32, (_TV, _D), 0)
    wt = jnp.where(row < _V - i * _TV, wt, 0.0)
    logits = lax.dot_general(
        e, wt, (((1,), (1,)), ((), ())), preferred_element_type=jnp.float32
    )
    a = jnp.abs(logits)
    z = jnp.exp2(a * (-1.4426950408889634))
    sp = 0.5 * (logits + a) + 0.6931471805599453 * jnp.log2(1.0 + z)
    tm = tm_ref[...]
    w_tok = (jnp.sum(tm, axis=1, keepdims=True) > 0).astype(jnp.float32)
    part = jnp.sum(jnp.sum(sp, axis=1, keepdims=True) * w_tok)

    @pl.when(i == 0)
    def _init():
        out_ref[...] = jnp.zeros((1, 1), jnp.float32)

    out_ref[...] += jnp.full((1, 1), part, jnp.float32)

    @pl.when(i == _NT - 1)
    def _fin():
        g = g_ref[...]
        lbl = lbl_ref[...]
        corr = jnp.float32(0.0)
        for p in range(_P):
            gp = g[p * _T : (p + 1) * _T, :]
            dots_lo = jnp.sum(gp[:, :_D] * e, axis=1, keepdims=True)
            dots_hi = jnp.sum(gp[:, _D:] * e, axis=1, keepdims=True)
            par = lbl[:, p : p + 1] % 2
            dots = jnp.where(par == 1, dots_hi, dots_lo)
            dp = w_tok
            for q in range(p):
                dp = dp * (lbl[:, p : p + 1] != lbl[:, q : q + 1]).astype(
                    jnp.float32
                )
            corr += jnp.sum(dots * dp)
        nv_sum = jnp.sum(w_tok)
        pad = jnp.float32(0.6931471805599453 * (_NT * _TV - _V)) * nv_sum
        nv = jnp.maximum(nv_sum, 1.0)
        out_ref[...] = (out_ref[...] - corr - pad) / nv


def kernel(model_embeddings, feature_tensors, positive_labels, negative_labels, padding_mask, target_padding_mask, W_items):
    e2 = model_embeddings.reshape(_T, _D)
    lbl2 = positive_labels.reshape(_T, _P).astype(jnp.int32)
    tm2 = target_padding_mask.reshape(_T, _P).astype(jnp.float32)
    idx_pm = lbl2.T.reshape(-1)
    idx_pad = jnp.concatenate(
        [idx_pm // 2, jnp.zeros((_GP - _P * _T,), jnp.int32)]
    )
    w_pairs = W_items.reshape(_V // 2, _D2)
    g = _make_sc_gather()(w_pairs, idx_pad)
    out = pl.pallas_call(
        _bce_body,
        grid=(_NT,),
        in_specs=[
            pl.BlockSpec((_T, _D), lambda i: (0, 0)),
            pl.BlockSpec((_TV, _D), lambda i: (i, 0)),
            pl.BlockSpec((_T, _P), lambda i: (0, 0)),
            pl.BlockSpec((_T, _P), lambda i: (0, 0)),
            pl.BlockSpec((_P * _T, _D2), lambda i: (0, 0)),
        ],
        out_specs=pl.BlockSpec((1, 1), lambda i: (0, 0)),
        out_shape=jax.ShapeDtypeStruct((1, 1), jnp.float32),
        compiler_params=pltpu.CompilerParams(
            dimension_semantics=("arbitrary",),
        ),
    )(e2, W_items, lbl2, tm2, g)
    return out[0, 0]
